# baseline (speedup 1.0000x reference)
import functools

import jax
import jax.numpy as jnp
from jax import lax
from jax.experimental import pallas as pl
from jax.experimental.pallas import tpu as pltpu
from jax.experimental.pallas import tpu_sc as plsc

_CHUNK_ROWS = 32


def _make_copy_kernel(seq_len: int, d_model: int):
    info = plsc.get_sparse_core_info()
    nc, ns = info.num_cores, info.num_subcores
    half = seq_len // nc
    rows_per_w = half // ns
    nchunks = rows_per_w // _CHUNK_ROWS
    mesh = plsc.VectorSubcoreMesh(core_axis_name="c", subcore_axis_name="s")

    @functools.partial(
        pl.kernel,
        out_type=(
            jax.ShapeDtypeStruct((seq_len, d_model), jnp.float32),
            jax.ShapeDtypeStruct((half, d_model), jnp.float32),
        ),
        mesh=mesh,
    )
    def copy_kernel(table_hbm, out0_hbm, out1_hbm):
        cid = lax.axis_index("c")
        sid = lax.axis_index("s")

        def scoped(buf0, buf1, si0, si1, so0, so1):
            bufs = (buf0, buf1)
            in_sems = (si0, si1)
            out_sems = (so0, so1)

            def pipeline(src_base, dst_ref, dst_base):
                def chunk_src(c):
                    return table_hbm.at[
                        pl.ds(src_base + c * _CHUNK_ROWS, _CHUNK_ROWS)
                    ]

                def chunk_dst(c):
                    return dst_ref.at[pl.ds(dst_base + c * _CHUNK_ROWS, _CHUNK_ROWS)]

                in_copies = [None] * nchunks
                out_copies = [None] * nchunks
                in_copies[0] = pltpu.async_copy(chunk_src(0), bufs[0], in_sems[0])
                for c in range(nchunks):
                    b = c % 2
                    in_copies[c].wait()
                    out_copies[c] = pltpu.async_copy(
                        bufs[b], chunk_dst(c), out_sems[b]
                    )
                    if c + 1 < nchunks:
                        if c >= 1:
                            out_copies[c - 1].wait()
                        nb = (c + 1) % 2
                        in_copies[c + 1] = pltpu.async_copy(
                            chunk_src(c + 1), bufs[nb], in_sems[nb]
                        )
                if nchunks >= 2:
                    out_copies[nchunks - 2].wait()
                out_copies[nchunks - 1].wait()

            @pl.when(cid == 0)
            def _():
                pipeline(sid * rows_per_w, out0_hbm, sid * rows_per_w)

            @pl.when(cid == 1)
            def _():
                pipeline(half + sid * rows_per_w, out1_hbm, sid * rows_per_w)

        pl.run_scoped(
            scoped,
            pltpu.VMEM((_CHUNK_ROWS, d_model), jnp.float32),
            pltpu.VMEM((_CHUNK_ROWS, d_model), jnp.float32),
            pltpu.SemaphoreType.DMA,
            pltpu.SemaphoreType.DMA,
            pltpu.SemaphoreType.DMA,
            pltpu.SemaphoreType.DMA,
        )

    return copy_kernel


def kernel(x, table):
    seq_len = x.shape[1]
    out0, out1 = _make_copy_kernel(seq_len, table.shape[1])(table)
    return lax.dynamic_update_slice(out0, out1, (seq_len // 2, 0))


# speedup vs baseline: 1.0596x; 1.0596x over previous
import functools

import jax
import jax.numpy as jnp
from jax import lax
from jax.experimental import pallas as pl
from jax.experimental.pallas import tpu as pltpu
from jax.experimental.pallas import tpu_sc as plsc

_CHUNK_ROWS = 32
_NBUF = 4


def _make_copy_kernel(seq_len: int, d_model: int):
    info = plsc.get_sparse_core_info()
    ns = info.num_subcores
    rows_per_w = seq_len // ns
    nchunks = rows_per_w // _CHUNK_ROWS
    mesh = plsc.VectorSubcoreMesh(
        core_axis_name="c", subcore_axis_name="s", num_cores=1
    )

    @functools.partial(
        pl.kernel,
        out_type=jax.ShapeDtypeStruct((seq_len, d_model), jnp.float32),
        mesh=mesh,
        scratch_types=(
            [pltpu.VMEM((_CHUNK_ROWS, d_model), jnp.float32) for _ in range(_NBUF)]
            + [pltpu.SemaphoreType.DMA for _ in range(2 * _NBUF)]
        ),
    )
    def copy_kernel(table_hbm, out_hbm, *scratch):
        bufs = scratch[:_NBUF]
        in_sems = scratch[_NBUF : 2 * _NBUF]
        out_sems = scratch[2 * _NBUF :]
        sid = lax.axis_index("s")
        base = sid * rows_per_w

        def chunk_src(c):
            return table_hbm.at[pl.ds(base + c * _CHUNK_ROWS, _CHUNK_ROWS)]

        def chunk_dst(c):
            return out_hbm.at[pl.ds(base + c * _CHUNK_ROWS, _CHUNK_ROWS)]

        in_copies = [None] * nchunks
        out_copies = [None] * nchunks
        for c in range(min(_NBUF, nchunks)):
            in_copies[c] = pltpu.async_copy(chunk_src(c), bufs[c], in_sems[c])
        for c in range(nchunks):
            b = c % _NBUF
            in_copies[c].wait()
            out_copies[c] = pltpu.async_copy(bufs[b], chunk_dst(c), out_sems[b])
            nxt = c + 1
            if nxt < nchunks and nxt >= _NBUF:
                out_copies[nxt - _NBUF].wait()
                nb = nxt % _NBUF
                in_copies[nxt] = pltpu.async_copy(
                    chunk_src(nxt), bufs[nb], in_sems[nb]
                )
        for c in range(max(0, nchunks - _NBUF), nchunks):
            out_copies[c].wait()

    return copy_kernel


def kernel(x, table):
    seq_len = x.shape[1]
    return _make_copy_kernel(seq_len, table.shape[1])(table)


# speedup vs baseline: 1.1791x; 1.1128x over previous
import functools

import jax
import jax.numpy as jnp
from jax import lax
from jax.experimental import pallas as pl
from jax.experimental.pallas import tpu as pltpu
from jax.experimental.pallas import tpu_sc as plsc

_CHUNK_ROWS = 32
_CORE0_ROWS = 2560


def _make_copy_kernel(seq_len: int, d_model: int):
    info = plsc.get_sparse_core_info()
    nc, ns = info.num_cores, info.num_subcores
    core_rows = (_CORE0_ROWS, seq_len - _CORE0_ROWS)
    core_base = (0, _CORE0_ROWS)
    mesh = plsc.VectorSubcoreMesh(core_axis_name="c", subcore_axis_name="s")

    @functools.partial(
        pl.kernel,
        out_type=jax.ShapeDtypeStruct((seq_len, d_model), jnp.float32),
        mesh=mesh,
        scratch_types=[
            pltpu.VMEM((_CHUNK_ROWS, d_model), jnp.float32),
            pltpu.VMEM((_CHUNK_ROWS, d_model), jnp.float32),
            pltpu.SemaphoreType.DMA,
            pltpu.SemaphoreType.DMA,
            pltpu.SemaphoreType.DMA,
            pltpu.SemaphoreType.DMA,
        ],
    )
    def copy_kernel(table_hbm, out_hbm, buf0, buf1, si0, si1, so0, so1):
        cid = lax.axis_index("c")
        sid = lax.axis_index("s")
        bufs = (buf0, buf1)
        in_sems = (si0, si1)
        out_sems = (so0, so1)

        def pipeline(base, nchunks):
            def chunk_src(c):
                return table_hbm.at[pl.ds(base + c * _CHUNK_ROWS, _CHUNK_ROWS)]

            def chunk_dst(c):
                return out_hbm.at[pl.ds(base + c * _CHUNK_ROWS, _CHUNK_ROWS)]

            in_copies = [None] * nchunks
            out_copies = [None] * nchunks
            in_copies[0] = pltpu.async_copy(chunk_src(0), bufs[0], in_sems[0])
            for c in range(nchunks):
                b = c % 2
                in_copies[c].wait()
                out_copies[c] = pltpu.async_copy(bufs[b], chunk_dst(c), out_sems[b])
                if c + 1 < nchunks:
                    if c >= 1:
                        out_copies[c - 1].wait()
                    nb = (c + 1) % 2
                    in_copies[c + 1] = pltpu.async_copy(
                        chunk_src(c + 1), bufs[nb], in_sems[nb]
                    )
            if nchunks >= 2:
                out_copies[nchunks - 2].wait()
            out_copies[nchunks - 1].wait()

        for core in range(nc):
            rows_per_w = core_rows[core] // ns

            @pl.when(cid == core)
            def _(core=core, rows_per_w=rows_per_w):
                pipeline(
                    core_base[core] + sid * rows_per_w,
                    rows_per_w // _CHUNK_ROWS,
                )

    return copy_kernel


def kernel(x, table):
    seq_len = x.shape[1]
    return _make_copy_kernel(seq_len, table.shape[1])(table)


# speedup vs baseline: 1.1929x; 1.0116x over previous
import functools

import jax
import jax.numpy as jnp
from jax import lax
from jax.experimental import pallas as pl
from jax.experimental.pallas import tpu as pltpu
from jax.experimental.pallas import tpu_sc as plsc

_CHUNK_ROWS = 32
_CORE0_ROWS = 1536


def _make_copy_kernel(seq_len: int, d_model: int):
    info = plsc.get_sparse_core_info()
    nc, ns = info.num_cores, info.num_subcores
    core_rows = (_CORE0_ROWS, seq_len - _CORE0_ROWS)
    core_base = (0, _CORE0_ROWS)
    mesh = plsc.VectorSubcoreMesh(core_axis_name="c", subcore_axis_name="s")

    @functools.partial(
        pl.kernel,
        out_type=jax.ShapeDtypeStruct((seq_len, d_model), jnp.float32),
        mesh=mesh,
        scratch_types=[
            pltpu.VMEM((_CHUNK_ROWS, d_model), jnp.float32),
            pltpu.VMEM((_CHUNK_ROWS, d_model), jnp.float32),
            pltpu.SemaphoreType.DMA,
            pltpu.SemaphoreType.DMA,
            pltpu.SemaphoreType.DMA,
            pltpu.SemaphoreType.DMA,
        ],
    )
    def copy_kernel(table_hbm, out_hbm, buf0, buf1, si0, si1, so0, so1):
        cid = lax.axis_index("c")
        sid = lax.axis_index("s")
        bufs = (buf0, buf1)
        in_sems = (si0, si1)
        out_sems = (so0, so1)

        def pipeline(base, nchunks):
            def chunk_src(c):
                return table_hbm.at[pl.ds(base + c * _CHUNK_ROWS, _CHUNK_ROWS)]

            def chunk_dst(c):
                return out_hbm.at[pl.ds(base + c * _CHUNK_ROWS, _CHUNK_ROWS)]

            in_copies = [None] * nchunks
            out_copies = [None] * nchunks
            in_copies[0] = pltpu.async_copy(chunk_src(0), bufs[0], in_sems[0])
            for c in range(nchunks):
                b = c % 2
                in_copies[c].wait()
                out_copies[c] = pltpu.async_copy(bufs[b], chunk_dst(c), out_sems[b])
                if c + 1 < nchunks:
                    if c >= 1:
                        out_copies[c - 1].wait()
                    nb = (c + 1) % 2
                    in_copies[c + 1] = pltpu.async_copy(
                        chunk_src(c + 1), bufs[nb], in_sems[nb]
                    )
            if nchunks >= 2:
                out_copies[nchunks - 2].wait()
            out_copies[nchunks - 1].wait()

        for core in range(nc):
            rows_per_w = core_rows[core] // ns

            @pl.when(cid == core)
            def _(core=core, rows_per_w=rows_per_w):
                pipeline(
                    core_base[core] + sid * rows_per_w,
                    rows_per_w // _CHUNK_ROWS,
                )

    return copy_kernel


def kernel(x, table):
    seq_len = x.shape[1]
    return _make_copy_kernel(seq_len, table.shape[1])(table)


# speedup vs baseline: 1.2439x; 1.0428x over previous
import functools

import jax
import jax.numpy as jnp
from jax import lax
from jax.experimental import pallas as pl
from jax.experimental.pallas import tpu as pltpu
from jax.experimental.pallas import tpu_sc as plsc

_CHUNK_ROWS = 32


def _make_copy_kernel(seq_len: int, d_model: int):
    info = plsc.get_sparse_core_info()
    nc, ns = info.num_cores, info.num_subcores
    nw = nc * ns
    rows_per_w = seq_len // nw
    nchunks = rows_per_w // _CHUNK_ROWS
    mesh = plsc.VectorSubcoreMesh(core_axis_name="c", subcore_axis_name="s")

    @functools.partial(
        pl.kernel,
        out_type=jax.ShapeDtypeStruct((seq_len, d_model), jnp.float32),
        mesh=mesh,
        scratch_types=[
            pltpu.VMEM((_CHUNK_ROWS, d_model), jnp.float32),
            pltpu.VMEM((_CHUNK_ROWS, d_model), jnp.float32),
            pltpu.SemaphoreType.DMA,
            pltpu.SemaphoreType.DMA,
            pltpu.SemaphoreType.DMA,
            pltpu.SemaphoreType.DMA,
        ],
    )
    def copy_kernel(table_hbm, out_hbm, buf0, buf1, si0, si1, so0, so1):
        wid = lax.axis_index("s") * nc + lax.axis_index("c")
        base = wid * rows_per_w
        bufs = (buf0, buf1)
        in_sems = (si0, si1)
        out_sems = (so0, so1)

        def chunk_src(c):
            return table_hbm.at[pl.ds(base + c * _CHUNK_ROWS, _CHUNK_ROWS)]

        def chunk_dst(c):
            return out_hbm.at[pl.ds(base + c * _CHUNK_ROWS, _CHUNK_ROWS)]

        in_copies = [None] * nchunks
        out_copies = [None] * nchunks
        in_copies[0] = pltpu.async_copy(chunk_src(0), bufs[0], in_sems[0])
        for c in range(nchunks):
            b = c % 2
            in_copies[c].wait()
            out_copies[c] = pltpu.async_copy(bufs[b], chunk_dst(c), out_sems[b])
            if c + 1 < nchunks:
                if c >= 1:
                    out_copies[c - 1].wait()
                nb = (c + 1) % 2
                in_copies[c + 1] = pltpu.async_copy(
                    chunk_src(c + 1), bufs[nb], in_sems[nb]
                )
        if nchunks >= 2:
            out_copies[nchunks - 2].wait()
        out_copies[nchunks - 1].wait()

    return copy_kernel


def kernel(x, table):
    seq_len = x.shape[1]
    return _make_copy_kernel(seq_len, table.shape[1])(table)
